# baseline (speedup 1.0000x reference)
import functools

import jax
import jax.numpy as jnp
from jax import lax
from jax.experimental import pallas as pl
from jax.experimental.pallas import tpu as pltpu
from jax.experimental.pallas import tpu_sc as plsc

B = 16384
F = 26
V = 1000000
K = 16
NW = 32
ROWS_W = B // NW
IDX_COLS = 128
IDXROWS_W = ROWS_W * F // IDX_COLS
CHUNK_BROWS = 128
CHUNK_G = CHUNK_BROWS * F // IDX_COLS
CHUNK_FLAT = CHUNK_BROWS * F
N_CHUNKS = ROWS_W // CHUNK_BROWS


def _fm_body(idx_hbm, vals_hbm, b_hbm, w_hbm, v_hbm, out_hbm,
             idx_v, vals_v, w_v, v_v, out_v, b_v, sem):
    wid = lax.axis_index("c") * 16 + lax.axis_index("s")

    pltpu.sync_copy(idx_hbm.at[pl.ds(wid * IDXROWS_W, IDXROWS_W)], idx_v)
    pltpu.sync_copy(b_hbm, b_v)
    b_vec = b_v[...]

    iota = lax.iota(jnp.int32, 16)
    zeros_i = jnp.zeros((16,), jnp.int32)
    hi_mask = jnp.where(iota >= 6, 1.0, 0.0).astype(jnp.float32)

    def chunk_body(c, carry):
        row0 = wid * ROWS_W + c * CHUNK_BROWS
        pltpu.sync_copy(vals_hbm.at[pl.ds(row0, CHUNK_BROWS)], vals_v)
        handles = []
        for g in range(CHUNK_G):
            isl = idx_v.at[c * CHUNK_G + g]
            handles.append(pltpu.async_copy(
                v_hbm.at[isl], v_v.at[pl.ds(g * IDX_COLS, IDX_COLS)], sem))
            handles.append(pltpu.async_copy(
                w_hbm.at[isl], w_v.at[pl.ds(g * IDX_COLS, IDX_COLS)], sem))
        for h in handles:
            h.wait()

        def rg_body(rg, inner_carry):
            res = jnp.zeros((16,), jnp.float32)
            for rl in range(16):
                r = rg * 16 + rl
                rbase = r * F
                vals0 = vals_v[r, 0:16]
                vals1 = vals_v[r, 10:26]
                acc = jnp.zeros((16,), jnp.float32)
                acc2 = jnp.zeros((16,), jnp.float32)
                for f in range(F):
                    rowv = v_v[rbase + f, :]
                    if f < 16:
                        vb = jnp.take_along_axis(
                            vals0, jnp.full((16,), f, jnp.int32), axis=0,
                            mode="promise_in_bounds")
                    else:
                        vb = jnp.take_along_axis(
                            vals1, jnp.full((16,), f - 10, jnp.int32), axis=0,
                            mode="promise_in_bounds")
                    t = rowv * vb
                    acc = acc + t
                    acc2 = acc2 + t * t
                wv0 = plsc.load_gather(w_v, [rbase + iota])
                wv1 = plsc.load_gather(w_v, [rbase + 10 + iota])
                linv = wv0 * vals0 + wv1 * vals1 * hi_mask
                s = 0.5 * jnp.sum(acc * acc - acc2) + jnp.sum(linv)
                res = jnp.where(iota == rl, jnp.full((16,), s, jnp.float32),
                                res)
            out_v[pl.ds((c * (CHUNK_BROWS // 16) + rg) * 16, 16)] = res + b_vec
            return inner_carry

        lax.fori_loop(0, CHUNK_BROWS // 16, rg_body, 0)
        return carry

    lax.fori_loop(0, N_CHUNKS, chunk_body, 0)
    pltpu.sync_copy(out_v, out_hbm.at[pl.ds(wid * ROWS_W, ROWS_W)])


NJ_FULL = V // 128
TAIL0 = NJ_FULL * 128
JBLK = 12
N_SLABS = NJ_FULL // JBLK
J_PER_W = (N_SLABS + NW - 1) // NW
SLAB_COLS = 128 * JBLK
SLAB_WORDS = SLAB_COLS * K


def _tr_body(vt_hbm, tail_hbm, out_hbm, tbuf, obuf):
    wid = lax.axis_index("c") * 16 + lax.axis_index("s")
    iota = lax.iota(jnp.int32, 16)

    def slab_body(i, carry):
        s = wid + i * NW
        @pl.when(s < N_SLABS)
        def _():
            c0 = s * SLAB_COLS
            pltpu.sync_copy(vt_hbm.at[:, pl.ds(c0, SLAB_COLS)], tbuf)

            def rg_body(rg, inner):
                for rl in range(16):
                    r = rg * 16 + rl
                    vec = plsc.load_gather(tbuf, [iota, jnp.full((16,), r,
                                                                 jnp.int32)])
                    obuf[pl.ds(r * 16, 16)] = vec
                return inner

            lax.fori_loop(0, SLAB_COLS // 16, rg_body, 0)
            pltpu.sync_copy(obuf, out_hbm.at[pl.ds(c0 * K, SLAB_WORDS)])
        return carry

    lax.fori_loop(0, J_PER_W, slab_body, 0)

    @pl.when(wid == 0)
    def _():
        pltpu.sync_copy(tail_hbm, obuf.at[pl.ds(0, (V - TAIL0) * K)])
        pltpu.sync_copy(obuf.at[pl.ds(0, (V - TAIL0) * K)],
                        out_hbm.at[pl.ds(TAIL0 * K, (V - TAIL0) * K)])


@jax.jit
def kernel(indices, values, b, w_weight, v_weight):
    idx2d = indices.reshape(B * F // IDX_COLS, IDX_COLS).astype(jnp.int32)
    b16 = jnp.broadcast_to(b.astype(jnp.float32), (16,))
    w1d = w_weight.T.reshape(V)
    vt = v_weight.T
    tail = v_weight[TAIL0:, :].reshape((V - TAIL0) * K)
    tr = pl.kernel(
        _tr_body,
        out_type=jax.ShapeDtypeStruct((V * K,), jnp.float32),
        mesh=plsc.VectorSubcoreMesh(core_axis_name="c", subcore_axis_name="s"),
        compiler_params=pltpu.CompilerParams(
            needs_layout_passes=False, use_tc_tiling_on_sc=True),
        scratch_types=[
            pltpu.VMEM((K, SLAB_COLS), jnp.float32),
            pltpu.VMEM((SLAB_WORDS,), jnp.float32),
        ],
    )
    v_lin = tr(vt, tail).reshape(V, K)
    fm = pl.kernel(
        _fm_body,
        out_type=jax.ShapeDtypeStruct((B,), jnp.float32),
        mesh=plsc.VectorSubcoreMesh(core_axis_name="c", subcore_axis_name="s"),
        compiler_params=pltpu.CompilerParams(
            needs_layout_passes=False, use_tc_tiling_on_sc=False),
        scratch_types=[
            pltpu.VMEM((IDXROWS_W, IDX_COLS), jnp.int32),
            pltpu.VMEM((CHUNK_BROWS, F), jnp.float32),
            pltpu.VMEM((CHUNK_FLAT,), jnp.float32),
            pltpu.VMEM((CHUNK_FLAT, K), jnp.float32),
            pltpu.VMEM((ROWS_W,), jnp.float32),
            pltpu.VMEM((16,), jnp.float32),
            pltpu.SemaphoreType.DMA,
        ],
    )
    return fm(idx2d, values, b16, w1d, v_lin)


# speedup vs baseline: 1.1099x; 1.1099x over previous
import functools

import jax
import jax.numpy as jnp
from jax import lax
from jax.experimental import pallas as pl
from jax.experimental.pallas import tpu as pltpu
from jax.experimental.pallas import tpu_sc as plsc

B = 16384
F = 26
V = 1000000
K = 16
NW = 32
ROWS_W = B // NW
IDX_COLS = 128
IDXROWS_W = ROWS_W * F // IDX_COLS
CHUNK_BROWS = 128
CHUNK_G = CHUNK_BROWS * F // IDX_COLS
CHUNK_FLAT = CHUNK_BROWS * F
N_CHUNKS = ROWS_W // CHUNK_BROWS


def _fm_body(idx_hbm, vals_hbm, b_hbm, w_hbm, v_hbm, out_hbm,
             idx_v, vals_v, w_v, v_v, out_v, b_v, sem):
    wid = lax.axis_index("c") * 16 + lax.axis_index("s")

    pltpu.sync_copy(idx_hbm.at[pl.ds(wid * IDXROWS_W, IDXROWS_W)], idx_v)
    pltpu.sync_copy(b_hbm, b_v)
    b_vec = b_v[...]

    iota = lax.iota(jnp.int32, 16)
    zeros_i = jnp.zeros((16,), jnp.int32)
    hi_mask = jnp.where(iota >= 6, 1.0, 0.0).astype(jnp.float32)

    def chunk_body(c, carry):
        row0 = wid * ROWS_W + c * CHUNK_BROWS
        pltpu.sync_copy(vals_hbm.at[pl.ds(row0, CHUNK_BROWS)], vals_v)
        handles = []
        for g in range(CHUNK_G):
            isl = idx_v.at[c * CHUNK_G + g]
            handles.append(pltpu.async_copy(
                v_hbm.at[isl], v_v.at[pl.ds(g * IDX_COLS, IDX_COLS)], sem))
            handles.append(pltpu.async_copy(
                w_hbm.at[isl], w_v.at[pl.ds(g * IDX_COLS, IDX_COLS)], sem))
        for h in handles:
            h.wait()

        def rg_body(rg, inner_carry):
            res = jnp.zeros((16,), jnp.float32)
            for rl in range(16):
                r = rg * 16 + rl
                rbase = r * F
                vals0 = vals_v[r, 0:16]
                vals1 = vals_v[r, 10:26]
                acc = jnp.zeros((16,), jnp.float32)
                acc2 = jnp.zeros((16,), jnp.float32)
                for f in range(F):
                    rowv = v_v[rbase + f, :]
                    if f < 16:
                        vb = jnp.take_along_axis(
                            vals0, jnp.full((16,), f, jnp.int32), axis=0,
                            mode="promise_in_bounds")
                    else:
                        vb = jnp.take_along_axis(
                            vals1, jnp.full((16,), f - 10, jnp.int32), axis=0,
                            mode="promise_in_bounds")
                    t = rowv * vb
                    acc = acc + t
                    acc2 = acc2 + t * t
                wv0 = plsc.load_gather(w_v, [rbase + iota])
                wv1 = plsc.load_gather(w_v, [rbase + 10 + iota])
                linv = wv0 * vals0 + wv1 * vals1 * hi_mask
                s = 0.5 * jnp.sum(acc * acc - acc2) + jnp.sum(linv)
                res = jnp.where(iota == rl, jnp.full((16,), s, jnp.float32),
                                res)
            out_v[pl.ds((c * (CHUNK_BROWS // 16) + rg) * 16, 16)] = res + b_vec
            return inner_carry

        lax.fori_loop(0, CHUNK_BROWS // 16, rg_body, 0)
        return carry

    lax.fori_loop(0, N_CHUNKS, chunk_body, 0)
    pltpu.sync_copy(out_v, out_hbm.at[pl.ds(wid * ROWS_W, ROWS_W)])


NJ_FULL = V // 128
TAIL0 = NJ_FULL * 128
JBLK = 12
N_SLABS = NJ_FULL // JBLK
J_PER_W = (N_SLABS + NW - 1) // NW
SLAB_COLS = 128 * JBLK
SLAB_WORDS = SLAB_COLS * K


def _tr_body(vt_hbm, tail_hbm, out_hbm,
             tbuf0, tbuf1, obuf0, obuf1, sin0, sin1, sout0, sout1):
    wid = lax.axis_index("c") * 16 + lax.axis_index("s")
    iota = lax.iota(jnp.int32, 16)
    tbufs, obufs = (tbuf0, tbuf1), (obuf0, obuf1)
    sins, souts = (sin0, sin1), (sout0, sout1)

    def slab_of(j):
        return wid + j * NW

    def start_in(j, b):
        @pl.when(slab_of(j) < N_SLABS)
        def _():
            pltpu.async_copy(
                vt_hbm.at[:, pl.ds(slab_of(j) * SLAB_COLS, SLAB_COLS)],
                tbufs[b], sins[b])

    def wait_in(j, b):
        @pl.when(slab_of(j) < N_SLABS)
        def _():
            pltpu.make_async_copy(
                vt_hbm.at[:, pl.ds(slab_of(j) * SLAB_COLS, SLAB_COLS)],
                tbufs[b], sins[b]).wait()

    def wait_out(j, b):
        @pl.when(jnp.logical_and(j >= 0, slab_of(j) < N_SLABS))
        def _():
            pltpu.make_async_copy(
                obufs[b],
                out_hbm.at[pl.ds(slab_of(j) * SLAB_WORDS, SLAB_WORDS)],
                souts[b]).wait()

    start_in(0, 0)
    start_in(1, 1)

    def slab_body(j, carry):
        def do(b):
            tb, ob = tbufs[b], obufs[b]
            wait_in(j, b)
            wait_out(j - 2, b)
            @pl.when(slab_of(j) < N_SLABS)
            def _():
                def rg_body(rg, inner):
                    bvec = jnp.full((16,), rg * 16, jnp.int32)
                    for rl in range(16):
                        vec = plsc.load_gather(tb, [iota, bvec + rl])
                        ob[pl.ds((rg * 16 + rl) * 16, 16)] = vec
                    return inner

                lax.fori_loop(0, SLAB_COLS // 16, rg_body, 0)
                pltpu.async_copy(
                    ob,
                    out_hbm.at[pl.ds(slab_of(j) * SLAB_WORDS, SLAB_WORDS)],
                    souts[b])
            start_in(j + 2, b)

        @pl.when(lax.rem(j, 2) == 0)
        def _():
            do(0)
        @pl.when(lax.rem(j, 2) == 1)
        def _():
            do(1)
        return carry

    lax.fori_loop(0, J_PER_W, slab_body, 0)
    wait_out(J_PER_W - 2, (J_PER_W - 2) % 2)
    wait_out(J_PER_W - 1, (J_PER_W - 1) % 2)

    @pl.when(wid == 0)
    def _():
        pltpu.sync_copy(tail_hbm, obuf0.at[pl.ds(0, (V - TAIL0) * K)])
        pltpu.sync_copy(obuf0.at[pl.ds(0, (V - TAIL0) * K)],
                        out_hbm.at[pl.ds(TAIL0 * K, (V - TAIL0) * K)])


@jax.jit
def kernel(indices, values, b, w_weight, v_weight):
    idx2d = indices.reshape(B * F // IDX_COLS, IDX_COLS).astype(jnp.int32)
    b16 = jnp.broadcast_to(b.astype(jnp.float32), (16,))
    w1d = w_weight.T.reshape(V)
    vt = v_weight.T
    tail = v_weight[TAIL0:, :].reshape((V - TAIL0) * K)
    tr = pl.kernel(
        _tr_body,
        out_type=jax.ShapeDtypeStruct((V * K,), jnp.float32),
        mesh=plsc.VectorSubcoreMesh(core_axis_name="c", subcore_axis_name="s"),
        compiler_params=pltpu.CompilerParams(
            needs_layout_passes=False, use_tc_tiling_on_sc=True),
        scratch_types=[
            pltpu.VMEM((K, SLAB_COLS), jnp.float32),
            pltpu.VMEM((K, SLAB_COLS), jnp.float32),
            pltpu.VMEM((SLAB_WORDS,), jnp.float32),
            pltpu.VMEM((SLAB_WORDS,), jnp.float32),
            pltpu.SemaphoreType.DMA,
            pltpu.SemaphoreType.DMA,
            pltpu.SemaphoreType.DMA,
            pltpu.SemaphoreType.DMA,
        ],
    )
    v_lin = tr(vt, tail).reshape(V, K)
    fm = pl.kernel(
        _fm_body,
        out_type=jax.ShapeDtypeStruct((B,), jnp.float32),
        mesh=plsc.VectorSubcoreMesh(core_axis_name="c", subcore_axis_name="s"),
        compiler_params=pltpu.CompilerParams(
            needs_layout_passes=False, use_tc_tiling_on_sc=False),
        scratch_types=[
            pltpu.VMEM((IDXROWS_W, IDX_COLS), jnp.int32),
            pltpu.VMEM((CHUNK_BROWS, F), jnp.float32),
            pltpu.VMEM((CHUNK_FLAT,), jnp.float32),
            pltpu.VMEM((CHUNK_FLAT, K), jnp.float32),
            pltpu.VMEM((ROWS_W,), jnp.float32),
            pltpu.VMEM((16,), jnp.float32),
            pltpu.SemaphoreType.DMA,
        ],
    )
    return fm(idx2d, values, b16, w1d, v_lin)
